# baseline (speedup 1.0000x reference)
import functools

import jax
import jax.numpy as jnp
from jax import lax
from jax.experimental import pallas as pl
from jax.experimental.pallas import tpu as pltpu
from jax.experimental.pallas import tpu_sc as plsc

MAXDEG = 8
DIM = 256
PF = 4
NROWS = 300
NPAD = 304
ROW_BLK = 8
B_TC = 9


def _idiv(a, n):
    return lax.div(a, jnp.full((16,), n, jnp.int32))


def _sc_gather_tile(edge_index, z_degree):
    info = plsc.get_sparse_core_info()
    nc = info.num_cores
    mesh = plsc.VectorSubcoreMesh(core_axis_name="c", subcore_axis_name="s")

    @functools.partial(
        pl.kernel,
        mesh=mesh,
        compiler_params=pltpu.CompilerParams(needs_layout_passes=False),
        out_type=jax.ShapeDtypeStruct((NPAD, DIM), jnp.float32),
        scratch_types=[
            pltpu.VMEM((48,), jnp.int32),
            pltpu.VMEM((32,), jnp.int32),
            pltpu.VMEM((128,), jnp.int32),
            pltpu.VMEM((128,), jnp.int32),
            pltpu.VMEM((48,), jnp.int32),
            pltpu.VMEM((128, DIM), jnp.float32),
            pltpu.VMEM((128, DIM), jnp.float32),
            pltpu.VMEM((48, DIM), jnp.float32),
            pltpu.SemaphoreType.DMA,
        ],
    )
    def sc_k(edge_hbm, z_hbm, out_hbm, src_v, deg_v, ia_v, ib_v, ic_v,
             ra_v, rb_v, rc_v, sem):
        wid = lax.axis_index("s") * nc + lax.axis_index("c")

        @pl.when(wid == 0)
        def _():
            pltpu.sync_copy(edge_hbm.at[0], src_v)
            zero16 = jnp.zeros((16,), jnp.int32)
            deg_v[pl.ds(0, 16)] = zero16
            deg_v[pl.ds(16, 16)] = zero16
            one16 = jnp.ones((16,), jnp.int32)
            for e in range(3):
                plsc.addupdate_scatter(deg_v, [src_v[pl.ds(e * 16, 16)]], one16)
            for ch in range(2):
                d = deg_v[pl.ds(ch * 16, 16)]
                d = jnp.maximum(jnp.minimum(d, MAXDEG) - 1, 0)
                deg_v[pl.ds(ch * 16, 16)] = d
            idx_bufs = ((ia_v, 0), (ib_v, 128), (ic_v, 256))
            for buf, base in idx_bufs:
                for ch in range(buf.shape[0] // 16):
                    r = lax.iota(jnp.int32, 16) + (base + ch * 16)
                    wh = _idiv(r, 20)
                    p1 = _idiv(r - wh * 20, 4)
                    j = (wh - _idiv(wh, 5) * 5) * 5 + p1
                    buf[pl.ds(ch * 16, 16)] = plsc.load_gather(deg_v, [j])
            cps = [
                pltpu.async_copy(z_hbm.at[ia_v], ra_v, sem),
                pltpu.async_copy(z_hbm.at[ib_v], rb_v, sem),
                pltpu.async_copy(z_hbm.at[ic_v], rc_v, sem),
            ]
            for cp in cps:
                cp.wait()
            pltpu.sync_copy(ra_v, out_hbm.at[pl.ds(0, 128)])
            pltpu.sync_copy(rb_v, out_hbm.at[pl.ds(128, 128)])
            pltpu.sync_copy(rc_v, out_hbm.at[pl.ds(256, 48)])

    return sc_k(edge_index, z_degree)


def _sc_broadcast(tile2, b, f):
    info = plsc.get_sparse_core_info()
    nc, ns = info.num_cores, info.num_subcores
    b_per_core = b // nc
    mesh = plsc.VectorSubcoreMesh(core_axis_name="c", subcore_axis_name="s")

    @functools.partial(
        pl.kernel,
        mesh=mesh,
        compiler_params=pltpu.CompilerParams(needs_layout_passes=False),
        out_type=jax.ShapeDtypeStruct((b, f, 15, 5120), jnp.float32),
        scratch_types=[
            pltpu.MemorySpace.VMEM_SHARED((16, 15, 5120), jnp.float32),
            pltpu.SemaphoreType.DMA,
            pltpu.SemaphoreType.DMA,
        ],
    )
    def k2(t_hbm, o_hbm, sp, isem, osem):
        cid = lax.axis_index("c")
        sid = lax.axis_index("s")
        pltpu.async_copy(t_hbm, sp.at[sid], isem).wait()
        plsc.subcore_barrier()
        bi = cid * b_per_core + sid // 2
        fj = (sid % 2) * 16
        pltpu.async_copy(sp, o_hbm.at[bi, pl.ds(fj, 16)], osem).wait()

    return k2(tile2)


def _tc_broadcast(tile2, part, b, f):
    nchunk = (B_TC * f) // ROW_BLK
    fchunk = f // ROW_BLK
    window = 8

    def body(t_hbm, part_hbm, o_hbm, t_v, insem, osem):
        fills = [pltpu.async_copy(t_hbm, t_v.at[k], insem)
                 for k in range(ROW_BLK)]
        for c in fills:
            c.wait()
        cps = []
        for i in range(nchunk):
            bi, fj = i // fchunk, (i % fchunk) * ROW_BLK
            cps.append(pltpu.async_copy(
                t_v, o_hbm.at[bi, pl.ds(fj, ROW_BLK)], osem))
            if i >= window:
                cps[i - window].wait()
        for c in cps[-window:]:
            c.wait()

    return pl.pallas_call(
        body,
        in_specs=[pl.BlockSpec(memory_space=pltpu.MemorySpace.HBM),
                  pl.BlockSpec(memory_space=pltpu.MemorySpace.HBM)],
        out_specs=pl.BlockSpec(memory_space=pltpu.MemorySpace.HBM),
        out_shape=jax.ShapeDtypeStruct((b, f, 15, 5120), jnp.float32),
        input_output_aliases={1: 0},
        scratch_shapes=[
            pltpu.VMEM((ROW_BLK, 15, 5120), jnp.float32),
            pltpu.SemaphoreType.DMA,
            pltpu.SemaphoreType.DMA,
        ],
    )(tile2, part)


def kernel(x, z_degree, edge_index):
    b, _, F, J, Wc = x.shape
    f = F // PF
    rows = _sc_gather_tile(edge_index, z_degree)
    tile = rows[:NROWS].reshape(15, 5120)
    return _sc_broadcast(tile, b, f)


# speedup vs baseline: 1.0264x; 1.0264x over previous
import functools

import jax
import jax.numpy as jnp
from jax import lax
from jax.experimental import pallas as pl
from jax.experimental.pallas import tpu as pltpu
from jax.experimental.pallas import tpu_sc as plsc

MAXDEG = 8
DIM = 256
PF = 4
NROWS = 300
NPAD = 304
ROW_BLK = 8
B_TC = 10


def _idiv(a, n):
    return lax.div(a, jnp.full((16,), n, jnp.int32))


def _sc_gather_tile(edge_index, z_degree):
    info = plsc.get_sparse_core_info()
    nc = info.num_cores
    mesh = plsc.VectorSubcoreMesh(core_axis_name="c", subcore_axis_name="s")

    @functools.partial(
        pl.kernel,
        mesh=mesh,
        compiler_params=pltpu.CompilerParams(needs_layout_passes=False),
        out_type=jax.ShapeDtypeStruct((NPAD, DIM), jnp.float32),
        scratch_types=[
            pltpu.VMEM((48,), jnp.int32),
            pltpu.VMEM((32,), jnp.int32),
            pltpu.VMEM((128,), jnp.int32),
            pltpu.VMEM((128,), jnp.int32),
            pltpu.VMEM((48,), jnp.int32),
            pltpu.VMEM((128, DIM), jnp.float32),
            pltpu.VMEM((128, DIM), jnp.float32),
            pltpu.VMEM((48, DIM), jnp.float32),
            pltpu.SemaphoreType.DMA,
        ],
    )
    def sc_k(edge_hbm, z_hbm, out_hbm, src_v, deg_v, ia_v, ib_v, ic_v,
             ra_v, rb_v, rc_v, sem):
        wid = lax.axis_index("s") * nc + lax.axis_index("c")

        @pl.when(wid == 0)
        def _():
            pltpu.sync_copy(edge_hbm.at[0], src_v)
            zero16 = jnp.zeros((16,), jnp.int32)
            deg_v[pl.ds(0, 16)] = zero16
            deg_v[pl.ds(16, 16)] = zero16
            one16 = jnp.ones((16,), jnp.int32)
            for e in range(3):
                plsc.addupdate_scatter(deg_v, [src_v[pl.ds(e * 16, 16)]], one16)
            for ch in range(2):
                d = deg_v[pl.ds(ch * 16, 16)]
                d = jnp.maximum(jnp.minimum(d, MAXDEG) - 1, 0)
                deg_v[pl.ds(ch * 16, 16)] = d
            idx_bufs = ((ia_v, 0), (ib_v, 128), (ic_v, 256))
            for buf, base in idx_bufs:
                for ch in range(buf.shape[0] // 16):
                    r = lax.iota(jnp.int32, 16) + (base + ch * 16)
                    wh = _idiv(r, 20)
                    p1 = _idiv(r - wh * 20, 4)
                    j = (wh - _idiv(wh, 5) * 5) * 5 + p1
                    buf[pl.ds(ch * 16, 16)] = plsc.load_gather(deg_v, [j])
            cps = [
                pltpu.async_copy(z_hbm.at[ia_v], ra_v, sem),
                pltpu.async_copy(z_hbm.at[ib_v], rb_v, sem),
                pltpu.async_copy(z_hbm.at[ic_v], rc_v, sem),
            ]
            for cp in cps:
                cp.wait()
            pltpu.sync_copy(ra_v, out_hbm.at[pl.ds(0, 128)])
            pltpu.sync_copy(rb_v, out_hbm.at[pl.ds(128, 128)])
            pltpu.sync_copy(rc_v, out_hbm.at[pl.ds(256, 48)])

    return sc_k(edge_index, z_degree)


def _sc_broadcast(tile2, b_sc, f):
    info = plsc.get_sparse_core_info()
    nc, ns = info.num_cores, info.num_subcores
    nw = nc * ns
    per_w = (b_sc * f) // nw
    window = 4
    mesh = plsc.VectorSubcoreMesh(core_axis_name="c", subcore_axis_name="s")

    @functools.partial(
        pl.kernel,
        mesh=mesh,
        compiler_params=pltpu.CompilerParams(needs_layout_passes=False),
        out_type=jax.ShapeDtypeStruct((b_sc, f, 15, 5120), jnp.float32),
        scratch_types=[
            pltpu.VMEM((15, 5120), jnp.float32),
            pltpu.SemaphoreType.DMA,
            pltpu.SemaphoreType.DMA,
        ],
    )
    def k2(t_hbm, o_hbm, tv, isem, osem):
        wid = lax.axis_index("s") * nc + lax.axis_index("c")
        pltpu.async_copy(t_hbm, tv, isem).wait()
        cps = []
        for kk in range(per_w):
            s = wid * per_w + kk
            cps.append(pltpu.async_copy(tv, o_hbm.at[s // f, s % f], osem))
            if kk >= window:
                cps[kk - window].wait()
        for c in cps[-window:]:
            c.wait()

    return k2(tile2)


def _tc_broadcast(tile2, b, f):
    nchunk = (B_TC * f) // ROW_BLK
    fchunk = f // ROW_BLK
    window = 8

    def body(t_hbm, o_hbm, t_v, insem, osem):
        fills = [pltpu.async_copy(t_hbm, t_v.at[k], insem)
                 for k in range(ROW_BLK)]
        for c in fills:
            c.wait()
        cps = []
        for i in range(nchunk):
            bi, fj = i // fchunk, (i % fchunk) * ROW_BLK
            cps.append(pltpu.async_copy(
                t_v, o_hbm.at[bi, pl.ds(fj, ROW_BLK)], osem))
            if i >= window:
                cps[i - window].wait()
        for c in cps[-window:]:
            c.wait()

    return pl.pallas_call(
        body,
        in_specs=[pl.BlockSpec(memory_space=pltpu.MemorySpace.HBM)],
        out_specs=pl.BlockSpec(memory_space=pltpu.MemorySpace.HBM),
        out_shape=jax.ShapeDtypeStruct((b, f, 15, 5120), jnp.float32),
        scratch_shapes=[
            pltpu.VMEM((ROW_BLK, 15, 5120), jnp.float32),
            pltpu.SemaphoreType.DMA,
            pltpu.SemaphoreType.DMA,
        ],
    )(tile2)


def kernel(x, z_degree, edge_index):
    b, _, F, J, Wc = x.shape
    f = F // PF
    rows = _sc_gather_tile(edge_index, z_degree)
    tile = rows[:NROWS].reshape(15, 5120)
    big = _tc_broadcast(tile, b, f)
    scp = _sc_broadcast(tile, b - B_TC, f)
    return lax.dynamic_update_slice(big, scp, (B_TC, 0, 0, 0))


# speedup vs baseline: 1.3026x; 1.2691x over previous
import functools

import jax
import jax.numpy as jnp
from jax import lax
from jax.experimental import pallas as pl
from jax.experimental.pallas import tpu as pltpu
from jax.experimental.pallas import tpu_sc as plsc

MAXDEG = 8
DIM = 256
PF = 4
NROWS = 300
NPAD = 304
ROW_BLK = 8
B_TC = 10


def _idiv(a, n):
    return lax.div(a, jnp.full((16,), n, jnp.int32))


def _sc_gather_tile(edge_index, z_degree):
    info = plsc.get_sparse_core_info()
    nc = info.num_cores
    mesh = plsc.VectorSubcoreMesh(core_axis_name="c", subcore_axis_name="s")

    @functools.partial(
        pl.kernel,
        mesh=mesh,
        compiler_params=pltpu.CompilerParams(needs_layout_passes=False),
        out_type=jax.ShapeDtypeStruct((NPAD, DIM), jnp.float32),
        scratch_types=[
            pltpu.VMEM((48,), jnp.int32),
            pltpu.VMEM((32,), jnp.int32),
            pltpu.VMEM((128,), jnp.int32),
            pltpu.VMEM((128,), jnp.int32),
            pltpu.VMEM((48,), jnp.int32),
            pltpu.VMEM((128, DIM), jnp.float32),
            pltpu.VMEM((128, DIM), jnp.float32),
            pltpu.VMEM((48, DIM), jnp.float32),
            pltpu.SemaphoreType.DMA,
        ],
    )
    def sc_k(edge_hbm, z_hbm, out_hbm, src_v, deg_v, ia_v, ib_v, ic_v,
             ra_v, rb_v, rc_v, sem):
        wid = lax.axis_index("s") * nc + lax.axis_index("c")

        @pl.when(wid == 0)
        def _():
            pltpu.sync_copy(edge_hbm.at[0], src_v)
            zero16 = jnp.zeros((16,), jnp.int32)
            deg_v[pl.ds(0, 16)] = zero16
            deg_v[pl.ds(16, 16)] = zero16
            one16 = jnp.ones((16,), jnp.int32)
            for e in range(3):
                plsc.addupdate_scatter(deg_v, [src_v[pl.ds(e * 16, 16)]], one16)
            for ch in range(2):
                d = deg_v[pl.ds(ch * 16, 16)]
                d = jnp.maximum(jnp.minimum(d, MAXDEG) - 1, 0)
                deg_v[pl.ds(ch * 16, 16)] = d
            idx_bufs = ((ia_v, 0), (ib_v, 128), (ic_v, 256))
            for buf, base in idx_bufs:
                for ch in range(buf.shape[0] // 16):
                    r = lax.iota(jnp.int32, 16) + (base + ch * 16)
                    wh = _idiv(r, 20)
                    p1 = _idiv(r - wh * 20, 4)
                    j = (wh - _idiv(wh, 5) * 5) * 5 + p1
                    buf[pl.ds(ch * 16, 16)] = plsc.load_gather(deg_v, [j])
            cps = [
                pltpu.async_copy(z_hbm.at[ia_v], ra_v, sem),
                pltpu.async_copy(z_hbm.at[ib_v], rb_v, sem),
                pltpu.async_copy(z_hbm.at[ic_v], rc_v, sem),
            ]
            for cp in cps:
                cp.wait()
            pltpu.sync_copy(ra_v, out_hbm.at[pl.ds(0, 128)])
            pltpu.sync_copy(rb_v, out_hbm.at[pl.ds(128, 128)])
            pltpu.sync_copy(rc_v, out_hbm.at[pl.ds(256, 48)])

    return sc_k(edge_index, z_degree)


def _sc_broadcast(tile2, b_sc, f):
    info = plsc.get_sparse_core_info()
    nc, ns = info.num_cores, info.num_subcores
    nw = nc * ns
    per_w = (b_sc * f) // nw
    window = 4
    mesh = plsc.VectorSubcoreMesh(core_axis_name="c", subcore_axis_name="s")

    @functools.partial(
        pl.kernel,
        mesh=mesh,
        compiler_params=pltpu.CompilerParams(needs_layout_passes=False),
        out_type=jax.ShapeDtypeStruct((b_sc, f, 15, 5120), jnp.float32),
        scratch_types=[
            pltpu.VMEM((15, 5120), jnp.float32),
            pltpu.SemaphoreType.DMA,
            pltpu.SemaphoreType.DMA,
        ],
    )
    def k2(t_hbm, o_hbm, tv, isem, osem):
        wid = lax.axis_index("s") * nc + lax.axis_index("c")
        pltpu.async_copy(t_hbm, tv, isem).wait()
        cps = []
        for kk in range(per_w):
            s = wid * per_w + kk
            cps.append(pltpu.async_copy(tv, o_hbm.at[s // f, s % f], osem))
            if kk >= window:
                cps[kk - window].wait()
        for c in cps[-window:]:
            c.wait()

    return k2(tile2)


def _tc_broadcast(tile2, b, f):
    nchunk = (b * f) // ROW_BLK
    window = 8

    def body(t_hbm, o_hbm, t_v, insem, osem):
        fills = [pltpu.async_copy(t_hbm, t_v.at[k], insem)
                 for k in range(ROW_BLK)]
        for c in fills:
            c.wait()
        cps = []
        for i in range(nchunk):
            cps.append(pltpu.async_copy(
                t_v, o_hbm.at[pl.ds(i * ROW_BLK, ROW_BLK)], osem))
            if i >= window:
                cps[i - window].wait()
        for c in cps[-window:]:
            c.wait()

    return pl.pallas_call(
        body,
        in_specs=[pl.BlockSpec(memory_space=pltpu.MemorySpace.HBM)],
        out_specs=pl.BlockSpec(memory_space=pltpu.MemorySpace.HBM),
        out_shape=jax.ShapeDtypeStruct((b * f, 15, 5120), jnp.float32),
        scratch_shapes=[
            pltpu.VMEM((ROW_BLK, 15, 5120), jnp.float32),
            pltpu.SemaphoreType.DMA,
            pltpu.SemaphoreType.DMA,
        ],
    )(tile2)


def kernel(x, z_degree, edge_index):
    b, _, F, J, Wc = x.shape
    f = F // PF
    rows = _sc_gather_tile(edge_index, z_degree)
    tile = rows[:NROWS].reshape(15, 5120)
    flat = _tc_broadcast(tile, b, f)
    return flat.reshape(b, f, 15, 5120)


# speedup vs baseline: 1.3058x; 1.0025x over previous
import functools

import jax
import jax.numpy as jnp
from jax import lax
from jax.experimental import pallas as pl
from jax.experimental.pallas import tpu as pltpu
from jax.experimental.pallas import tpu_sc as plsc

MAXDEG = 8
DIM = 256
PF = 4
NROWS = 300
NPAD = 304
ROW_BLK = 8
B_TC = 10


def _idiv(a, n):
    return lax.div(a, jnp.full((16,), n, jnp.int32))


def _sc_gather_tile(edge_index, z_degree):
    info = plsc.get_sparse_core_info()
    nc = info.num_cores
    mesh = plsc.VectorSubcoreMesh(core_axis_name="c", subcore_axis_name="s")

    @functools.partial(
        pl.kernel,
        mesh=mesh,
        compiler_params=pltpu.CompilerParams(needs_layout_passes=False),
        out_type=jax.ShapeDtypeStruct((NPAD, DIM), jnp.float32),
        scratch_types=[
            pltpu.VMEM((48,), jnp.int32),
            pltpu.VMEM((32,), jnp.int32),
            pltpu.VMEM((128,), jnp.int32),
            pltpu.VMEM((128,), jnp.int32),
            pltpu.VMEM((48,), jnp.int32),
            pltpu.VMEM((128, DIM), jnp.float32),
            pltpu.VMEM((128, DIM), jnp.float32),
            pltpu.VMEM((48, DIM), jnp.float32),
            pltpu.SemaphoreType.DMA,
        ],
    )
    def sc_k(edge_hbm, z_hbm, out_hbm, src_v, deg_v, ia_v, ib_v, ic_v,
             ra_v, rb_v, rc_v, sem):
        wid = lax.axis_index("s") * nc + lax.axis_index("c")

        @pl.when(wid == 0)
        def _():
            pltpu.sync_copy(edge_hbm.at[0], src_v)
            zero16 = jnp.zeros((16,), jnp.int32)
            deg_v[pl.ds(0, 16)] = zero16
            deg_v[pl.ds(16, 16)] = zero16
            one16 = jnp.ones((16,), jnp.int32)
            for e in range(3):
                plsc.addupdate_scatter(deg_v, [src_v[pl.ds(e * 16, 16)]], one16)
            for ch in range(2):
                d = deg_v[pl.ds(ch * 16, 16)]
                d = jnp.maximum(jnp.minimum(d, MAXDEG) - 1, 0)
                deg_v[pl.ds(ch * 16, 16)] = d
            idx_bufs = ((ia_v, 0), (ib_v, 128), (ic_v, 256))
            for buf, base in idx_bufs:
                for ch in range(buf.shape[0] // 16):
                    r = lax.iota(jnp.int32, 16) + (base + ch * 16)
                    wh = _idiv(r, 20)
                    p1 = _idiv(r - wh * 20, 4)
                    j = (wh - _idiv(wh, 5) * 5) * 5 + p1
                    buf[pl.ds(ch * 16, 16)] = plsc.load_gather(deg_v, [j])
            cps = [
                pltpu.async_copy(z_hbm.at[ia_v], ra_v, sem),
                pltpu.async_copy(z_hbm.at[ib_v], rb_v, sem),
                pltpu.async_copy(z_hbm.at[ic_v], rc_v, sem),
            ]
            for cp in cps:
                cp.wait()
            pltpu.sync_copy(ra_v, out_hbm.at[pl.ds(0, 128)])
            pltpu.sync_copy(rb_v, out_hbm.at[pl.ds(128, 128)])
            pltpu.sync_copy(rc_v, out_hbm.at[pl.ds(256, 48)])

    return sc_k(edge_index, z_degree)


def _sc_broadcast(tile2, b_sc, f):
    info = plsc.get_sparse_core_info()
    nc, ns = info.num_cores, info.num_subcores
    nw = nc * ns
    per_w = (b_sc * f) // nw
    window = 4
    mesh = plsc.VectorSubcoreMesh(core_axis_name="c", subcore_axis_name="s")

    @functools.partial(
        pl.kernel,
        mesh=mesh,
        compiler_params=pltpu.CompilerParams(needs_layout_passes=False),
        out_type=jax.ShapeDtypeStruct((b_sc, f, 15, 5120), jnp.float32),
        scratch_types=[
            pltpu.VMEM((15, 5120), jnp.float32),
            pltpu.SemaphoreType.DMA,
            pltpu.SemaphoreType.DMA,
        ],
    )
    def k2(t_hbm, o_hbm, tv, isem, osem):
        wid = lax.axis_index("s") * nc + lax.axis_index("c")
        pltpu.async_copy(t_hbm, tv, isem).wait()
        cps = []
        for kk in range(per_w):
            s = wid * per_w + kk
            cps.append(pltpu.async_copy(tv, o_hbm.at[s // f, s % f], osem))
            if kk >= window:
                cps[kk - window].wait()
        for c in cps[-window:]:
            c.wait()

    return k2(tile2)


def _tc_broadcast(tile2, b, f):
    nchunk = (b * f) // ROW_BLK
    window = 8

    def body(t_hbm, o_hbm, t_v, insem, osem):
        fills = [pltpu.async_copy(t_hbm, t_v.at[k], insem)
                 for k in range(ROW_BLK)]
        for c in fills:
            c.wait()
        cps = []
        for i in range(nchunk):
            cps.append(pltpu.async_copy(
                t_v, o_hbm.at[pl.ds(i * ROW_BLK, ROW_BLK)], osem))
            if i >= window:
                cps[i - window].wait()
        for c in cps[-window:]:
            c.wait()

    return pl.pallas_call(
        body,
        in_specs=[pl.BlockSpec(memory_space=pltpu.MemorySpace.HBM)],
        out_specs=pl.BlockSpec(memory_space=pltpu.MemorySpace.HBM),
        out_shape=jax.ShapeDtypeStruct((b * f, 15, 5120), jnp.float32),
        scratch_shapes=[
            pltpu.VMEM((ROW_BLK, 15, 5120), jnp.float32),
            pltpu.SemaphoreType.DMA,
            pltpu.SemaphoreType.DMA,
        ],
    )(tile2)


def kernel(x, z_degree, edge_index):
    b, _, F, J, Wc = x.shape
    f = F // PF
    rows = _sc_gather_tile(edge_index, z_degree)
    tile = rows[:NROWS].reshape(15, 5120)
    bc = 2
    flat = _tc_broadcast(tile, bc, f)
    rs = flat.reshape(bc, f, 15, 5120)
    return jnp.concatenate([rs] * (b // bc), axis=0)
